# baseline (speedup 1.0000x reference)
import functools

import jax
import jax.numpy as jnp
from jax import lax
from jax.experimental import pallas as pl
from jax.experimental.pallas import tpu as pltpu
from jax.experimental.pallas import tpu_sc as plsc

_DIM = 32
_B = 16384
_NC = 2
_NS = 16
_NW = _NC * _NS

_NCLS = 11
_NREL = 2
_NG = _NCLS + _NREL
_CLS_PER_W = _NCLS * _B // _NW
_REL_PER_W = _NREL * _B // _NW
_CH_CLS = 352
_CH_REL = 256
_NCH_CLS = _CLS_PER_W // _CH_CLS
_NCH_REL = _REL_PER_W // _CH_REL


def _tr_body(a_ref, out_ref):
    a = a_ref[...]
    eye = (lax.broadcasted_iota(jnp.int32, (a.shape[0],) * 2, 0)
           == lax.broadcasted_iota(jnp.int32, (a.shape[0],) * 2, 1)
           ).astype(jnp.float32)
    y = lax.dot_general(a, eye, (((0,), (0,)), ((), ())),
                        preferred_element_type=jnp.float32)
    blk = out_ref.shape[0]
    pieces = [y[m * blk:(m + 1) * blk, :] for m in range(2048 // blk)]
    out_ref[...] = jnp.concatenate(pieces, axis=1)


def _pack_table(emb):
    n, d = emb.shape
    v = emb.T
    blk = 2048 * d // 128
    grid = (n + 2047) // 2048
    return pl.pallas_call(
        _tr_body,
        grid=(grid,),
        in_specs=[pl.BlockSpec((d, 2048), lambda i: (0, i))],
        out_specs=pl.BlockSpec((blk, 128), lambda i: (i, 0)),
        out_shape=jax.ShapeDtypeStruct((grid * blk, 128), jnp.float32),
    )(v)


@functools.cache
def _make_sc_gather():
    @functools.partial(
        pl.kernel,
        mesh=plsc.VectorSubcoreMesh(core_axis_name="c", subcore_axis_name="s"),
        out_type=jax.ShapeDtypeStruct((_NG * _B, 128), jnp.float32),
        scratch_types=[
            pltpu.VMEM((_CH_CLS,), jnp.int32),
            pltpu.VMEM((_CH_CLS,), jnp.int32),
            pltpu.VMEM((_CH_CLS, 128), jnp.float32),
            pltpu.VMEM((_CH_CLS, 128), jnp.float32),
            pltpu.SemaphoreType.DMA,
            pltpu.SemaphoreType.DMA,
        ],
    )
    def _sc_gather(idx_h, cls_h, rel_h, out_h,
                   idx_a, idx_b, rows_a, rows_b, sem_a, sem_b):
        wid = lax.axis_index("s") * _NC + lax.axis_index("c")

        chunks = []
        cbase = wid * _CLS_PER_W
        for k in range(_NCH_CLS):
            chunks.append((cls_h, cbase + k * _CH_CLS, _CH_CLS))
        rbase = _NCLS * _B + wid * _REL_PER_W
        for k in range(_NCH_REL):
            chunks.append((rel_h, rbase + k * _CH_REL, _CH_REL))

        bufs = [(idx_a, rows_a, sem_a), (idx_b, rows_b, sem_b)]
        handles = [None, None]

        def start(k):
            tab, o, n = chunks[k]
            i_v, r_v, s_v = bufs[k % 2]
            pltpu.sync_copy(idx_h.at[pl.ds(o, n)], i_v.at[pl.ds(0, n)])
            handles[k % 2] = pltpu.async_copy(
                tab.at[i_v.at[pl.ds(0, n)]], r_v.at[pl.ds(0, n)], s_v)

        start(0)
        for k in range(len(chunks)):
            if k + 1 < len(chunks):
                start(k + 1)
            _, o, n = chunks[k]
            _, r_v, _ = bufs[k % 2]
            handles[k % 2].wait()
            pltpu.sync_copy(r_v.at[pl.ds(0, n)], out_h.at[pl.ds(o, n)])

    return _sc_gather


_BB = 1024


def _tc_loss_body(rows_ref, *rest):
    idx_refs = rest[:_NG]
    wr_ref, out_ref = rest[_NG], rest[_NG + 1]
    d = _DIM

    def halves(k):
        g = rows_ref[k]
        b = ((idx_refs[k][...] & 1).reshape(_BB, 1)) != 0
        row = jnp.where(b, g[:, 64:128], g[:, 0:64])
        return row[:, :d], row[:, d:]

    def rel(k):
        g = rows_ref[_NCLS + k]
        s = idx_refs[_NCLS + k][...]
        b0 = ((s & 1).reshape(_BB, 1)) != 0
        b1 = ((s & 2).reshape(_BB, 1)) != 0
        w01 = jnp.where(b0, g[:, 32:64], g[:, 0:32])
        w23 = jnp.where(b0, g[:, 96:128], g[:, 64:96])
        return jnp.where(b1, w23, w01)

    def nrm(x):
        t = jnp.maximum(x, 0.0)
        return jnp.sqrt(jnp.sum(t * t, axis=1))

    def mm(x):
        return jnp.dot(x, wr_ref[...], preferred_element_type=jnp.float32)

    cC, cO = halves(0)
    dC, dO = halves(1)
    s = nrm(dC - cC) + nrm(cO - dO) + nrm(cC - cO) + nrm(dC - dO)

    cC, cO = halves(2)
    dC, dO = halves(3)
    eC, eO = halves(4)
    startAll = jnp.maximum(cC, dC)
    endAll = jnp.minimum(cO, dO)
    s += (nrm(eC - startAll) + nrm(endAll - eO) + nrm(cC - cO)
          + nrm(dC - dO) + nrm(eC - eO))

    a, b = halves(5)
    p, q = halves(6)
    rC = rel(0)
    cC = mm(a) + rC
    cO = mm(b) + rC
    dC = mm(p)
    dO = mm(q)
    s += nrm(dC - cC) + nrm(cO - dO) + nrm(dC - dO) + nrm(cC - cO)

    a, b = halves(7)
    p, q = halves(8)
    rC = rel(1)
    cC = mm(a)
    cO = mm(b)
    dC = mm(p) + rC
    dO = mm(q) + rC
    s += nrm(dC - cC) + nrm(cO - dO) + nrm(dC - dO) + nrm(cC - cO)

    cC, cO = halves(9)
    dC, dO = halves(10)
    startAll = jnp.maximum(cC, dC)
    endAll = jnp.minimum(cO, dO)
    s += nrm(endAll - startAll) + nrm(cC - cO) + nrm(dC - dO)

    @pl.when(pl.program_id(0) == 0)
    def _():
        out_ref[...] = jnp.zeros((1, 1), jnp.float32)

    out_ref[...] += (jnp.sum(s) * (1.0 / _B)).reshape(1, 1)


def kernel(nf1, nf2, nf3, nf4, disjoint, classEmb, relEmb, Wr):
    idx_cols = [
        nf1[:, 0], nf1[:, 1],
        nf2[:, 0], nf2[:, 1], nf2[:, 2],
        nf3[:, 0], nf3[:, 2],
        nf4[:, 1], nf4[:, 2],
        disjoint[:, 0], disjoint[:, 1],
        nf3[:, 1], nf4[:, 0],
    ]
    idx_cols = [c.astype(jnp.int32) for c in idx_cols]
    cls_piece = [(c >> 10) & 1 for c in idx_cols[:_NCLS]]
    rel_piece = [(c >> 9) & 3 for c in idx_cols[_NCLS:]]
    idx_all = jnp.concatenate(
        [((c >> 11) << 10) | (c & 1023) for c in idx_cols[:_NCLS]]
        + [((c >> 11) << 9) | (c & 511) for c in idx_cols[_NCLS:]])
    piece_cols = cls_piece + rel_piece

    cls128 = _pack_table(classEmb)
    rel128 = _pack_table(relEmb)

    rows = _make_sc_gather()(idx_all, cls128, rel128)
    rows = rows.reshape(_NG, _B, 128)

    out = pl.pallas_call(
        _tc_loss_body,
        grid=(_B // _BB,),
        in_specs=[pl.BlockSpec((_NG, _BB, 128), lambda i: (0, i, 0))]
        + [pl.BlockSpec((_BB,), lambda i: (i,)) for _ in range(_NG)]
        + [pl.BlockSpec((_DIM, _DIM), lambda i: (0, 0))],
        out_specs=pl.BlockSpec((1, 1), lambda i: (0, 0)),
        out_shape=jax.ShapeDtypeStruct((1, 1), jnp.float32),
    )(rows, *piece_cols, Wr)
    return out[0, 0]


# speedup vs baseline: 1.5134x; 1.5134x over previous
import functools

import jax
import jax.numpy as jnp
from jax import lax
from jax.experimental import pallas as pl
from jax.experimental.pallas import tpu as pltpu
from jax.experimental.pallas import tpu_sc as plsc

_DIM = 32
_B = 16384
_NC = 2
_NS = 16
_NW = _NC * _NS

_NCLS = 11
_NREL = 2
_NG = _NCLS + _NREL
_CLS_PER_W = _NCLS * _B // _NW
_REL_PER_W = _NREL * _B // _NW
_CH_CLS = 352
_CH_REL = 256
_NCH_CLS = _CLS_PER_W // _CH_CLS
_NCH_REL = _REL_PER_W // _CH_REL


def _tr_body(a_ref, out_ref):
    a = a_ref[...]
    eye = (lax.broadcasted_iota(jnp.int32, (a.shape[0],) * 2, 0)
           == lax.broadcasted_iota(jnp.int32, (a.shape[0],) * 2, 1)
           ).astype(jnp.float32)
    y = lax.dot_general(a, eye, (((0,), (0,)), ((), ())),
                        preferred_element_type=jnp.float32)
    blk = out_ref.shape[0]
    pieces = [y[m * blk:(m + 1) * blk, :] for m in range(_W // blk)]
    out_ref[...] = jnp.concatenate(pieces, axis=1)


_W = 16384


def _pack_table(emb):
    n, d = emb.shape
    v = emb.T
    blk = _W * d // 128
    grid = (n + _W - 1) // _W
    return pl.pallas_call(
        _tr_body,
        grid=(grid,),
        in_specs=[pl.BlockSpec((d, _W), lambda i: (0, i))],
        out_specs=pl.BlockSpec((blk, 128), lambda i: (i, 0)),
        out_shape=jax.ShapeDtypeStruct((grid * blk, 128), jnp.float32),
    )(v)


@functools.cache
def _make_sc_gather():
    @functools.partial(
        pl.kernel,
        mesh=plsc.VectorSubcoreMesh(core_axis_name="c", subcore_axis_name="s"),
        out_type=jax.ShapeDtypeStruct((_NG * _B, 128), jnp.float32),
        scratch_types=[
            pltpu.VMEM((_CH_CLS,), jnp.int32),
            pltpu.VMEM((_CH_CLS,), jnp.int32),
            pltpu.VMEM((_CH_CLS, 128), jnp.float32),
            pltpu.VMEM((_CH_CLS, 128), jnp.float32),
            pltpu.SemaphoreType.DMA,
            pltpu.SemaphoreType.DMA,
        ],
    )
    def _sc_gather(idx_h, cls_h, rel_h, out_h,
                   idx_a, idx_b, rows_a, rows_b, sem_a, sem_b):
        wid = lax.axis_index("s") * _NC + lax.axis_index("c")

        chunks = []
        cbase = wid * _CLS_PER_W
        for k in range(_NCH_CLS):
            chunks.append((cls_h, cbase + k * _CH_CLS, _CH_CLS))
        rbase = _NCLS * _B + wid * _REL_PER_W
        for k in range(_NCH_REL):
            chunks.append((rel_h, rbase + k * _CH_REL, _CH_REL))

        bufs = [(idx_a, rows_a, sem_a), (idx_b, rows_b, sem_b)]
        handles = [None, None]

        def start(k):
            tab, o, n = chunks[k]
            i_v, r_v, s_v = bufs[k % 2]
            pltpu.sync_copy(idx_h.at[pl.ds(o, n)], i_v.at[pl.ds(0, n)])
            handles[k % 2] = pltpu.async_copy(
                tab.at[i_v.at[pl.ds(0, n)]], r_v.at[pl.ds(0, n)], s_v)

        start(0)
        for k in range(len(chunks)):
            if k + 1 < len(chunks):
                start(k + 1)
            _, o, n = chunks[k]
            _, r_v, _ = bufs[k % 2]
            handles[k % 2].wait()
            pltpu.sync_copy(r_v.at[pl.ds(0, n)], out_h.at[pl.ds(o, n)])

    return _sc_gather


_BB = 1024


def _tc_loss_body(rows_ref, *rest):
    idx_refs = rest[:_NG]
    wr_ref, out_ref = rest[_NG], rest[_NG + 1]
    d = _DIM

    def halves(k):
        g = rows_ref[k]
        b = ((idx_refs[k][...] & 1).reshape(_BB, 1)) != 0
        row = jnp.where(b, g[:, 64:128], g[:, 0:64])
        return row[:, :d], row[:, d:]

    def rel(k):
        g = rows_ref[_NCLS + k]
        s = idx_refs[_NCLS + k][...]
        b0 = ((s & 1).reshape(_BB, 1)) != 0
        b1 = ((s & 2).reshape(_BB, 1)) != 0
        w01 = jnp.where(b0, g[:, 32:64], g[:, 0:32])
        w23 = jnp.where(b0, g[:, 96:128], g[:, 64:96])
        return jnp.where(b1, w23, w01)

    def nrm(x):
        t = jnp.maximum(x, 0.0)
        return jnp.sqrt(jnp.sum(t * t, axis=1))

    def mm(x):
        return jnp.dot(x, wr_ref[...], preferred_element_type=jnp.float32)

    cC, cO = halves(0)
    dC, dO = halves(1)
    s = nrm(dC - cC) + nrm(cO - dO) + nrm(cC - cO) + nrm(dC - dO)

    cC, cO = halves(2)
    dC, dO = halves(3)
    eC, eO = halves(4)
    startAll = jnp.maximum(cC, dC)
    endAll = jnp.minimum(cO, dO)
    s += (nrm(eC - startAll) + nrm(endAll - eO) + nrm(cC - cO)
          + nrm(dC - dO) + nrm(eC - eO))

    a, b = halves(5)
    p, q = halves(6)
    rC = rel(0)
    cC = mm(a) + rC
    cO = mm(b) + rC
    dC = mm(p)
    dO = mm(q)
    s += nrm(dC - cC) + nrm(cO - dO) + nrm(dC - dO) + nrm(cC - cO)

    a, b = halves(7)
    p, q = halves(8)
    rC = rel(1)
    cC = mm(a)
    cO = mm(b)
    dC = mm(p) + rC
    dO = mm(q) + rC
    s += nrm(dC - cC) + nrm(cO - dO) + nrm(dC - dO) + nrm(cC - cO)

    cC, cO = halves(9)
    dC, dO = halves(10)
    startAll = jnp.maximum(cC, dC)
    endAll = jnp.minimum(cO, dO)
    s += nrm(endAll - startAll) + nrm(cC - cO) + nrm(dC - dO)

    @pl.when(pl.program_id(0) == 0)
    def _():
        out_ref[...] = jnp.zeros((1, 1), jnp.float32)

    out_ref[...] += (jnp.sum(s) * (1.0 / _B)).reshape(1, 1)


def kernel(nf1, nf2, nf3, nf4, disjoint, classEmb, relEmb, Wr):
    idx_cols = [
        nf1[:, 0], nf1[:, 1],
        nf2[:, 0], nf2[:, 1], nf2[:, 2],
        nf3[:, 0], nf3[:, 2],
        nf4[:, 1], nf4[:, 2],
        disjoint[:, 0], disjoint[:, 1],
        nf3[:, 1], nf4[:, 0],
    ]
    idx_cols = [c.astype(jnp.int32) for c in idx_cols]
    cls_piece = [(c >> 13) & 1 for c in idx_cols[:_NCLS]]
    rel_piece = [(c >> 12) & 3 for c in idx_cols[_NCLS:]]
    idx_all = jnp.concatenate(
        [((c >> 14) << 13) | (c & 8191) for c in idx_cols[:_NCLS]]
        + [((c >> 14) << 12) | (c & 4095) for c in idx_cols[_NCLS:]])
    piece_cols = cls_piece + rel_piece

    cls128 = _pack_table(classEmb)
    rel128 = _pack_table(relEmb)

    rows = _make_sc_gather()(idx_all, cls128, rel128)
    rows = rows.reshape(_NG, _B, 128)

    out = pl.pallas_call(
        _tc_loss_body,
        grid=(_B // _BB,),
        in_specs=[pl.BlockSpec((_NG, _BB, 128), lambda i: (0, i, 0))]
        + [pl.BlockSpec((_BB,), lambda i: (i,)) for _ in range(_NG)]
        + [pl.BlockSpec((_DIM, _DIM), lambda i: (0, 0))],
        out_specs=pl.BlockSpec((1, 1), lambda i: (0, 0)),
        out_shape=jax.ShapeDtypeStruct((1, 1), jnp.float32),
    )(rows, *piece_cols, Wr)
    return out[0, 0]


# speedup vs baseline: 2.1276x; 1.4058x over previous
import functools

import jax
import jax.numpy as jnp
from jax import lax
from jax.experimental import pallas as pl
from jax.experimental.pallas import tpu as pltpu
from jax.experimental.pallas import tpu_sc as plsc

_DIM = 32
_B = 16384
_NC = 2
_NS = 16
_NW = _NC * _NS

_NCLS = 11
_NREL = 2
_NG = _NCLS + _NREL
_CLS_PER_W = _NCLS * _B // _NW
_REL_PER_W = _NREL * _B // _NW
_CH_CLS = 352
_CH_REL = 256
_NCH_CLS = _CLS_PER_W // _CH_CLS
_NCH_REL = _REL_PER_W // _CH_REL


def _tr_body(a_ref, out_ref):
    a = a_ref[...]
    d, w = a.shape
    blk = out_ref.shape[0]
    a2 = jnp.concatenate(
        [a[:, m * blk:(m + 1) * blk] for m in range(w // blk)], axis=0)
    eye = (lax.broadcasted_iota(jnp.int32, (128, 128), 0)
           == lax.broadcasted_iota(jnp.int32, (128, 128), 1)
           ).astype(jnp.float32)
    out_ref[...] = lax.dot_general(a2, eye, (((0,), (0,)), ((), ())),
                                   preferred_element_type=jnp.float32)


def _pack_table(emb, win):
    n, d = emb.shape
    v = emb.T
    blk = win * d // 128
    grid = (n + win - 1) // win
    return pl.pallas_call(
        _tr_body,
        grid=(grid,),
        in_specs=[pl.BlockSpec((d, win), lambda i: (0, i))],
        out_specs=pl.BlockSpec((blk, 128), lambda i: (i, 0)),
        out_shape=jax.ShapeDtypeStruct((grid * blk, 128), jnp.float32),
    )(v)


@functools.cache
def _make_sc_gather(total_rows, chunk):
    per_w = total_rows // _NW
    nch = per_w // chunk

    @functools.partial(
        pl.kernel,
        mesh=plsc.VectorSubcoreMesh(core_axis_name="c", subcore_axis_name="s"),
        out_type=jax.ShapeDtypeStruct((total_rows, 128), jnp.float32),
        scratch_types=[
            pltpu.VMEM((chunk,), jnp.int32),
            pltpu.VMEM((chunk,), jnp.int32),
            pltpu.VMEM((chunk, 128), jnp.float32),
            pltpu.VMEM((chunk, 128), jnp.float32),
            pltpu.SemaphoreType.DMA,
            pltpu.SemaphoreType.DMA,
        ],
    )
    def _sc_gather(idx_h, tab_h, out_h,
                   idx_a, idx_b, rows_a, rows_b, sem_a, sem_b):
        wid = lax.axis_index("s") * _NC + lax.axis_index("c")
        base = wid * per_w
        bufs = [(idx_a, rows_a, sem_a), (idx_b, rows_b, sem_b)]
        handles = [None, None]

        def start(k):
            i_v, r_v, s_v = bufs[k % 2]
            pltpu.sync_copy(idx_h.at[pl.ds(base + k * chunk, chunk)], i_v)
            handles[k % 2] = pltpu.async_copy(tab_h.at[i_v], r_v, s_v)

        start(0)
        for k in range(nch):
            if k + 1 < nch:
                start(k + 1)
            _, r_v, _ = bufs[k % 2]
            handles[k % 2].wait()
            pltpu.sync_copy(r_v, out_h.at[pl.ds(base + k * chunk, chunk)])

    return _sc_gather


_BB = 1024


def _tc_loss_body(crows_ref, rrows_ref, *rest):
    idx_refs = rest[:_NG]
    wr_ref, out_ref = rest[_NG], rest[_NG + 1]
    d = _DIM

    def halves(k):
        g = crows_ref[k]
        b = ((idx_refs[k][...] & 1).reshape(_BB, 1)) != 0
        row = jnp.where(b, g[:, 64:128], g[:, 0:64])
        return row[:, :d], row[:, d:]

    def rel(k):
        g = rrows_ref[k]
        s = idx_refs[_NCLS + k][...]
        b0 = ((s & 1).reshape(_BB, 1)) != 0
        b1 = ((s & 2).reshape(_BB, 1)) != 0
        w01 = jnp.where(b0, g[:, 32:64], g[:, 0:32])
        w23 = jnp.where(b0, g[:, 96:128], g[:, 64:96])
        return jnp.where(b1, w23, w01)

    def nrm(x):
        t = jnp.maximum(x, 0.0)
        return jnp.sqrt(jnp.sum(t * t, axis=1))

    def mm(x):
        return jnp.dot(x, wr_ref[...], preferred_element_type=jnp.float32)

    cC, cO = halves(0)
    dC, dO = halves(1)
    s = nrm(dC - cC) + nrm(cO - dO) + nrm(cC - cO) + nrm(dC - dO)

    cC, cO = halves(2)
    dC, dO = halves(3)
    eC, eO = halves(4)
    startAll = jnp.maximum(cC, dC)
    endAll = jnp.minimum(cO, dO)
    s += (nrm(eC - startAll) + nrm(endAll - eO) + nrm(cC - cO)
          + nrm(dC - dO) + nrm(eC - eO))

    a, b = halves(5)
    p, q = halves(6)
    rC = rel(0)
    cC = mm(a) + rC
    cO = mm(b) + rC
    dC = mm(p)
    dO = mm(q)
    s += nrm(dC - cC) + nrm(cO - dO) + nrm(dC - dO) + nrm(cC - cO)

    a, b = halves(7)
    p, q = halves(8)
    rC = rel(1)
    cC = mm(a)
    cO = mm(b)
    dC = mm(p) + rC
    dO = mm(q) + rC
    s += nrm(dC - cC) + nrm(cO - dO) + nrm(dC - dO) + nrm(cC - cO)

    cC, cO = halves(9)
    dC, dO = halves(10)
    startAll = jnp.maximum(cC, dC)
    endAll = jnp.minimum(cO, dO)
    s += nrm(endAll - startAll) + nrm(cC - cO) + nrm(dC - dO)

    @pl.when(pl.program_id(0) == 0)
    def _():
        out_ref[...] = jnp.zeros((1, 1), jnp.float32)

    out_ref[...] += (jnp.sum(s) * (1.0 / _B)).reshape(1, 1)


def kernel(nf1, nf2, nf3, nf4, disjoint, classEmb, relEmb, Wr):
    idx_cols = [
        nf1[:, 0], nf1[:, 1],
        nf2[:, 0], nf2[:, 1], nf2[:, 2],
        nf3[:, 0], nf3[:, 2],
        nf4[:, 1], nf4[:, 2],
        disjoint[:, 0], disjoint[:, 1],
        nf3[:, 1], nf4[:, 0],
    ]
    idx_cols = [c.astype(jnp.int32) for c in idx_cols]
    cls_piece = [(c >> 13) & 1 for c in idx_cols[:_NCLS]]
    rel_piece = [(c >> 13) & 3 for c in idx_cols[_NCLS:]]
    cls_idx = jnp.concatenate(
        [((c >> 14) << 13) | (c & 8191) for c in idx_cols[:_NCLS]])
    rel_idx = jnp.concatenate(
        [((c >> 15) << 13) | (c & 8191) for c in idx_cols[_NCLS:]])
    piece_cols = cls_piece + rel_piece

    cls128 = _pack_table(classEmb, 16384)
    crows = _make_sc_gather(_NCLS * _B, _CH_CLS)(cls_idx, cls128)
    rel128 = _pack_table(relEmb, 32768)
    rrows = _make_sc_gather(_NREL * _B, _CH_REL)(rel_idx, rel128)
    crows = crows.reshape(_NCLS, _B, 128)
    rrows = rrows.reshape(_NREL, _B, 128)

    out = pl.pallas_call(
        _tc_loss_body,
        grid=(_B // _BB,),
        in_specs=[pl.BlockSpec((_NCLS, _BB, 128), lambda i: (0, i, 0)),
                  pl.BlockSpec((_NREL, _BB, 128), lambda i: (0, i, 0))]
        + [pl.BlockSpec((_BB,), lambda i: (i,)) for _ in range(_NG)]
        + [pl.BlockSpec((_DIM, _DIM), lambda i: (0, 0))],
        out_specs=pl.BlockSpec((1, 1), lambda i: (0, 0)),
        out_shape=jax.ShapeDtypeStruct((1, 1), jnp.float32),
    )(crows, rrows, *piece_cols, Wr)
    return out[0, 0]


# speedup vs baseline: 2.3549x; 1.1068x over previous
import functools

import jax
import jax.numpy as jnp
from jax import lax
from jax.experimental import pallas as pl
from jax.experimental.pallas import tpu as pltpu
from jax.experimental.pallas import tpu_sc as plsc

_DIM = 32
_B = 16384
_NC = 2
_NS = 16
_NW = _NC * _NS

_NCLS = 11
_NREL = 2
_NG = _NCLS + _NREL
_CLS_PER_W = _NCLS * _B // _NW
_REL_PER_W = _NREL * _B // _NW
_CH_CLS = 352
_CH_REL = 256
_NCH_CLS = _CLS_PER_W // _CH_CLS
_NCH_REL = _REL_PER_W // _CH_REL


def _tr_body(a_ref, out_ref):
    a = a_ref[...]
    d, w = a.shape
    blk = out_ref.shape[0]
    a2 = jnp.concatenate(
        [a[:, m * blk:(m + 1) * blk] for m in range(w // blk)], axis=0)
    eye = (lax.broadcasted_iota(jnp.int32, (128, 128), 0)
           == lax.broadcasted_iota(jnp.int32, (128, 128), 1)
           ).astype(jnp.float32)
    out_ref[...] = lax.dot_general(a2, eye, (((0,), (0,)), ((), ())),
                                   preferred_element_type=jnp.float32)


def _pack_table(emb, win):
    n, d = emb.shape
    v = emb.T
    blk = win * d // 128
    grid = (n + win - 1) // win
    return pl.pallas_call(
        _tr_body,
        grid=(grid,),
        in_specs=[pl.BlockSpec((d, win), lambda i: (0, i))],
        out_specs=pl.BlockSpec((blk, 128), lambda i: (i, 0)),
        out_shape=jax.ShapeDtypeStruct((grid * blk, 128), jnp.float32),
    )(v)


@functools.cache
def _make_sc_gather(total_rows, chunk):
    per_w = total_rows // _NW
    nch = per_w // chunk

    @functools.partial(
        pl.kernel,
        mesh=plsc.VectorSubcoreMesh(core_axis_name="c", subcore_axis_name="s"),
        out_type=jax.ShapeDtypeStruct((total_rows, 128), jnp.float32),
        scratch_types=[
            pltpu.VMEM((chunk,), jnp.int32),
            pltpu.VMEM((chunk,), jnp.int32),
            pltpu.VMEM((chunk, 128), jnp.float32),
            pltpu.VMEM((chunk, 128), jnp.float32),
            pltpu.SemaphoreType.DMA,
            pltpu.SemaphoreType.DMA,
        ],
    )
    def _sc_gather(idx_h, tab_h, out_h,
                   idx_a, idx_b, rows_a, rows_b, sem_a, sem_b):
        wid = lax.axis_index("s") * _NC + lax.axis_index("c")
        base = wid * per_w
        bufs = [(idx_a, rows_a, sem_a), (idx_b, rows_b, sem_b)]
        handles = [None, None]

        def start(k):
            i_v, r_v, s_v = bufs[k % 2]
            pltpu.sync_copy(idx_h.at[pl.ds(base + k * chunk, chunk)], i_v)
            handles[k % 2] = pltpu.async_copy(tab_h.at[i_v], r_v, s_v)

        start(0)
        for k in range(nch):
            if k + 1 < nch:
                start(k + 1)
            _, r_v, _ = bufs[k % 2]
            handles[k % 2].wait()
            pltpu.sync_copy(r_v, out_h.at[pl.ds(base + k * chunk, chunk)])

    return _sc_gather


_BB = 2048


def _tc_loss_body(crows_ref, rrows_ref, *rest):
    idx_refs = rest[:_NG]
    w4_ref, m_ref, out_ref = rest[_NG], rest[_NG + 1], rest[_NG + 2]
    d = _DIM

    def sel(k):
        g = crows_ref[k]
        b = ((idx_refs[k][...] & 1).reshape(_BB, 1)) != 0
        return jnp.where(b, g[:, 64:128], g[:, 0:64])

    def halves(k):
        row = sel(k)
        return row[:, :d], row[:, d:]

    def rel(k):
        g = rrows_ref[k]
        s = idx_refs[_NCLS + k][...]
        b0 = ((s & 1).reshape(_BB, 1)) != 0
        b1 = ((s & 2).reshape(_BB, 1)) != 0
        w01 = jnp.where(b0, g[:, 32:64], g[:, 0:32])
        w23 = jnp.where(b0, g[:, 96:128], g[:, 64:96])
        return jnp.where(b1, w23, w01)

    def mm4(r1, r2):
        y = jnp.dot(jnp.concatenate([r1, r2], axis=1), w4_ref[...],
                    preferred_element_type=jnp.float32)
        return y[:, 0:32], y[:, 32:64], y[:, 64:96], y[:, 96:128]

    terms = []

    cC, cO = halves(0)
    dC, dO = halves(1)
    terms += [dC - cC, cO - dO, cC - cO, dC - dO]

    cC, cO = halves(2)
    dC, dO = halves(3)
    eC, eO = halves(4)
    terms += [eC - jnp.maximum(cC, dC), jnp.minimum(cO, dO) - eO,
              cC - cO, dC - dO, eC - eO]

    aW, bW, pW, qW = mm4(sel(5), sel(6))
    rC = rel(0)
    cC, cO, dC, dO = aW + rC, bW + rC, pW, qW
    terms += [dC - cC, cO - dO, dC - dO, cC - cO]

    aW, bW, pW, qW = mm4(sel(7), sel(8))
    rC = rel(1)
    cC, cO, dC, dO = aW, bW, pW + rC, qW + rC
    terms += [dC - cC, cO - dO, dC - dO, cC - cO]

    cC, cO = halves(9)
    dC, dO = halves(10)
    terms += [jnp.minimum(cO, dO) - jnp.maximum(cC, dC), cC - cO, dC - dO]

    t = jnp.maximum(jnp.concatenate(terms, axis=1), 0.0)
    s2 = jnp.dot(t * t, m_ref[...], preferred_element_type=jnp.float32)
    total = jnp.sum(jnp.sqrt(s2))

    @pl.when(pl.program_id(0) == 0)
    def _():
        out_ref[...] = jnp.zeros((1, 1), jnp.float32)

    out_ref[...] += (total * (1.0 / _B)).reshape(1, 1)


def kernel(nf1, nf2, nf3, nf4, disjoint, classEmb, relEmb, Wr):
    idx_cols = [
        nf1[:, 0], nf1[:, 1],
        nf2[:, 0], nf2[:, 1], nf2[:, 2],
        nf3[:, 0], nf3[:, 2],
        nf4[:, 1], nf4[:, 2],
        disjoint[:, 0], disjoint[:, 1],
        nf3[:, 1], nf4[:, 0],
    ]
    idx_cols = [c.astype(jnp.int32) for c in idx_cols]
    cls_piece = [(c >> 14) & 1 for c in idx_cols[:_NCLS]]
    rel_piece = [(c >> 14) & 3 for c in idx_cols[_NCLS:]]
    cls_idx = jnp.concatenate(
        [((c >> 15) << 14) | (c & 16383) for c in idx_cols[:_NCLS]])
    rel_idx = jnp.concatenate(
        [((c >> 16) << 14) | (c & 16383) for c in idx_cols[_NCLS:]])
    piece_cols = cls_piece + rel_piece

    w4 = jnp.kron(jnp.eye(4, dtype=jnp.float32), Wr.astype(jnp.float32))
    msum = (jnp.arange(640)[:, None] // 32
            == jnp.arange(128)[None, :]).astype(jnp.float32)

    cls128 = _pack_table(classEmb, 32768)
    crows = _make_sc_gather(_NCLS * _B, _CH_CLS)(cls_idx, cls128)
    rel128 = _pack_table(relEmb, 65536)
    rrows = _make_sc_gather(_NREL * _B, _CH_REL)(rel_idx, rel128)
    crows = crows.reshape(_NCLS, _B, 128)
    rrows = rrows.reshape(_NREL, _B, 128)

    out = pl.pallas_call(
        _tc_loss_body,
        grid=(_B // _BB,),
        in_specs=[pl.BlockSpec((_NCLS, _BB, 128), lambda i: (0, i, 0)),
                  pl.BlockSpec((_NREL, _BB, 128), lambda i: (0, i, 0))]
        + [pl.BlockSpec((_BB,), lambda i: (i,)) for _ in range(_NG)]
        + [pl.BlockSpec((128, 128), lambda i: (0, 0)),
           pl.BlockSpec((640, 128), lambda i: (0, 0))],
        out_specs=pl.BlockSpec((1, 1), lambda i: (0, 0)),
        out_shape=jax.ShapeDtypeStruct((1, 1), jnp.float32),
    )(crows, rrows, *piece_cols, w4, msum)
    return out[0, 0]
